# Initial kernel scaffold; baseline (speedup 1.0000x reference)
import jax
import jax.numpy as jnp
from jax.experimental import pallas as pl


def kernel(rig_t_world, camera_t_rig, projection, image_idx):
    raise NotImplementedError("write your pallas kernel here")


# speedup vs baseline: 3.8805x; 3.8805x over previous
import functools

import jax
import jax.numpy as jnp
from jax import lax
from jax.experimental import pallas as pl
from jax.experimental.pallas import tpu as pltpu
from jax.experimental.pallas import tpu_sc as plsc

NC = 2
NS = 16
L = 16
NW = NC * NS
CHUNK = 128
NCHUNK = 4
BPW = CHUNK * NCHUNK


def _sc_body(rig_hbm, cam_hbm, projpad_hbm, idx_hbm, cw_out, proj_out,
             idx_v, fid_v, cid_v, rrows_v, crows_v, prows_v, cw_v, sem):
    wid = lax.axis_index("s") * NC + lax.axis_index("c")

    pltpu.sync_copy(idx_hbm.at[wid], idx_v)

    for c in range(NCHUNK):
        for i in range(CHUNK // L):
            v = idx_v[c, pl.ds(i * L, L)]
            fid_v[c, pl.ds(i * L, L)] = lax.shift_right_logical(v, 3)
            cid_v[c, pl.ds(i * L, L)] = lax.bitwise_and(v, 7)

    handles = []
    for c in range(NCHUNK):
        handles.append(pltpu.async_copy(rig_hbm.at[fid_v.at[c]], rrows_v.at[c], sem))
        handles.append(pltpu.async_copy(cam_hbm.at[cid_v.at[c]], crows_v.at[c], sem))
        handles.append(pltpu.async_copy(projpad_hbm.at[cid_v.at[c]], prows_v.at[c], sem))
    for h in handles:
        h.wait()

    iota = lax.iota(jnp.int32, L)
    idx_a = [lax.bitwise_and(iota, 12) + j for j in range(4)]
    idx_r = [lax.bitwise_and(iota, 3) + 4 * j for j in range(4)]

    def take16(vec, idx):
        return lax.gather(
            vec, idx[:, None],
            lax.GatherDimensionNumbers(
                offset_dims=(), collapsed_slice_dims=(0,),
                start_index_map=(0,)),
            (1,), mode=lax.GatherScatterMode.PROMISE_IN_BOUNDS)

    for c in range(NCHUNK):

        def body(b, _, c=c):
            crow = crows_v[c, b]
            rrow = rrows_v[c, b]
            acc = None
            for j in range(4):
                term = take16(crow, idx_a[j]) * take16(rrow, idx_r[j])
                acc = term if acc is None else acc + term
            cw_v[c, b] = acc
            return 0

        lax.fori_loop(0, CHUNK, body, 0, unroll=4)

    pltpu.sync_copy(cw_v, cw_out.at[wid])
    pltpu.sync_copy(prows_v, proj_out.at[wid])


@functools.partial(jax.jit, static_argnums=())
def _sc_call(rig, cam, projpad, idx4):
    mesh = plsc.VectorSubcoreMesh(core_axis_name="c", subcore_axis_name="s")
    f = pl.kernel(
        _sc_body,
        out_type=[
            jax.ShapeDtypeStruct((NW, NCHUNK, CHUNK, L), jnp.float32),
            jax.ShapeDtypeStruct((NW, NCHUNK, CHUNK, L), jnp.float32),
        ],
        mesh=mesh,
        scratch_types=[
            pltpu.VMEM((NCHUNK, CHUNK), jnp.int32),
            pltpu.VMEM((NCHUNK, CHUNK), jnp.int32),
            pltpu.VMEM((NCHUNK, CHUNK), jnp.int32),
            pltpu.VMEM((NCHUNK, CHUNK, L), jnp.float32),
            pltpu.VMEM((NCHUNK, CHUNK, L), jnp.float32),
            pltpu.VMEM((NCHUNK, CHUNK, L), jnp.float32),
            pltpu.VMEM((NCHUNK, CHUNK, L), jnp.float32),
            pltpu.SemaphoreType.DMA,
        ],
        compiler_params=pltpu.CompilerParams(use_tc_tiling_on_sc=False),
    )
    return f(rig, cam, projpad, idx4)


def kernel(rig_t_world, camera_t_rig, projection, image_idx):
    nf = rig_t_world.shape[0]
    ncam = camera_t_rig.shape[0]
    b = image_idx.shape[0]

    rig = rig_t_world.reshape(nf, 16)
    cam = camera_t_rig.reshape(ncam, 16)
    projpad = jnp.concatenate(
        [projection.reshape(ncam, 9),
         jnp.zeros((ncam, 7), jnp.float32)], axis=1)
    idx4 = image_idx.reshape(NW, NCHUNK, CHUNK)

    cw4, proj4 = _sc_call(rig, cam, projpad, idx4)

    camera_t_world = cw4.reshape(b, 4, 4)
    proj = proj4.reshape(b, 16)[:, :9].reshape(b, 3, 3)
    return (camera_t_world, proj)


# speedup vs baseline: 6.5462x; 1.6870x over previous
import functools

import jax
import jax.numpy as jnp
from jax import lax
from jax.experimental import pallas as pl
from jax.experimental.pallas import tpu as pltpu
from jax.experimental.pallas import tpu_sc as plsc

NC = 2
NS = 16
L = 16
NW = NC * NS
CHUNK = 128
NCHUNK = 4
BPW = CHUNK * NCHUNK


def _take16(vec, idx):
    return lax.gather(
        vec, idx[:, None],
        lax.GatherDimensionNumbers(
            offset_dims=(), collapsed_slice_dims=(0,),
            start_index_map=(0,)),
        (1,), mode=lax.GatherScatterMode.PROMISE_IN_BOUNDS)


def _sc_body(rig_hbm, cam_hbm, proj_hbm, idx_hbm, cw_out, proj_out,
             idx_v, fid_v, cid_v, rrows_v, camtab_v, projtab_v,
             cw_v, prj_v, sem):
    wid = lax.axis_index("s") * NC + lax.axis_index("c")

    pltpu.sync_copy(idx_hbm.at[wid], idx_v)

    for c in range(NCHUNK):
        for i in range(CHUNK // L):
            v = idx_v[c, pl.ds(i * L, L)]
            fid_v[c, pl.ds(i * L, L)] = lax.shift_right_logical(v, 3)
            cid_v[c, pl.ds(i * L, L)] = lax.bitwise_and(v, 7)

    handles = [
        pltpu.async_copy(rig_hbm.at[fid_v.at[c]], rrows_v.at[c], sem)
        for c in range(NCHUNK)
    ]
    pltpu.sync_copy(cam_hbm, camtab_v)
    pltpu.sync_copy(proj_hbm, projtab_v)

    iota = lax.iota(jnp.int32, L)
    idx_a = [lax.bitwise_and(iota, 12) + j for j in range(4)]
    idx_r = [lax.bitwise_and(iota, 3) + 4 * j for j in range(4)]
    evecs = [jnp.full((L,), e, dtype=jnp.int32) for e in range(L)]

    for c in range(NCHUNK):
        handles[c].wait()

        def group(g, _, c=c):
            cidv = cid_v[c, pl.ds(g * L, L)]
            for e in range(L):
                b = g * L + e
                cb16 = lax.shift_left(_take16(cidv, evecs[e]), 4)
                rrow = rrows_v[c, b]
                prj_v[c, b] = plsc.load_gather(projtab_v, [cb16 + iota])
                acc = None
                for j in range(4):
                    aj = plsc.load_gather(camtab_v, [cb16 + idx_a[j]])
                    term = aj * _take16(rrow, idx_r[j])
                    acc = term if acc is None else acc + term
                cw_v[c, b] = acc
            return 0

        lax.fori_loop(0, CHUNK // L, group, 0)

    pltpu.sync_copy(cw_v, cw_out.at[wid])
    pltpu.sync_copy(prj_v, proj_out.at[wid])


def _sc_call(rig, cam, proj, idx4):
    mesh = plsc.VectorSubcoreMesh(core_axis_name="c", subcore_axis_name="s")
    f = pl.kernel(
        _sc_body,
        out_type=[
            jax.ShapeDtypeStruct((NW, NCHUNK, CHUNK, L), jnp.float32),
            jax.ShapeDtypeStruct((NW, NCHUNK, CHUNK, L), jnp.float32),
        ],
        mesh=mesh,
        scratch_types=[
            pltpu.VMEM((NCHUNK, CHUNK), jnp.int32),
            pltpu.VMEM((NCHUNK, CHUNK), jnp.int32),
            pltpu.VMEM((NCHUNK, CHUNK), jnp.int32),
            pltpu.VMEM((NCHUNK, CHUNK, L), jnp.float32),
            pltpu.VMEM((8 * L,), jnp.float32),
            pltpu.VMEM((8 * L,), jnp.float32),
            pltpu.VMEM((NCHUNK, CHUNK, L), jnp.float32),
            pltpu.VMEM((NCHUNK, CHUNK, L), jnp.float32),
            pltpu.SemaphoreType.DMA,
        ],
        compiler_params=pltpu.CompilerParams(
            use_tc_tiling_on_sc=False, needs_layout_passes=False),
    )
    return f(rig, cam, proj, idx4)


def kernel(rig_t_world, camera_t_rig, projection, image_idx):
    nf = rig_t_world.shape[0]
    ncam = camera_t_rig.shape[0]
    b = image_idx.shape[0]

    rig = rig_t_world.reshape(nf, 16)
    cam = camera_t_rig.reshape(ncam * 16)
    projpad = jnp.concatenate(
        [projection.reshape(ncam, 9),
         jnp.zeros((ncam, 7), jnp.float32)], axis=1).reshape(ncam * 16)
    idx4 = image_idx.reshape(NW, NCHUNK, CHUNK)

    cw4, proj4 = _sc_call(rig, cam, projpad, idx4)

    camera_t_world = cw4.reshape(b, 4, 4)
    proj = proj4.reshape(b, 16)[:, :9].reshape(b, 3, 3)
    return (camera_t_world, proj)


# speedup vs baseline: 7.5724x; 1.1568x over previous
import jax
import jax.numpy as jnp
from jax import lax
from jax.experimental import pallas as pl
from jax.experimental.pallas import tpu as pltpu
from jax.experimental.pallas import tpu_sc as plsc

NC = 2
NS = 16
L = 16
NW = NC * NS
CHUNK = 128
NCHUNK = 4
BPW = CHUNK * NCHUNK


def _take16(vec, idx):
    return lax.gather(
        vec, idx[:, None],
        lax.GatherDimensionNumbers(
            offset_dims=(), collapsed_slice_dims=(0,),
            start_index_map=(0,)),
        (1,), mode=lax.GatherScatterMode.PROMISE_IN_BOUNDS)


def _sc_body(rig_hbm, cam_hbm, proj_hbm, idx_hbm, cw_out, proj_out,
             idx_v, fid_v, cid_v, rrows_v, camtab_v, projtab_v,
             cwT_v, prjT_v, sem):
    wid = lax.axis_index("s") * NC + lax.axis_index("c")

    pltpu.sync_copy(idx_hbm.at[wid], idx_v)

    for c in range(NCHUNK):
        for i in range(CHUNK // L):
            v = idx_v[c, pl.ds(i * L, L)]
            fid_v[c, pl.ds(i * L, L)] = lax.shift_right_logical(v, 3)
            cid_v[c, pl.ds(i * L, L)] = lax.bitwise_and(v, 7)

    handles = [
        pltpu.async_copy(rig_hbm.at[fid_v.at[c]], rrows_v.at[c], sem)
        for c in range(NCHUNK)
    ]
    pltpu.sync_copy(cam_hbm, camtab_v)
    pltpu.sync_copy(proj_hbm, projtab_v)

    iota = lax.iota(jnp.int32, L)
    idx_a = [lax.bitwise_and(iota, 12) + j for j in range(4)]
    idx_r = [lax.bitwise_and(iota, 3) + 4 * j for j in range(4)]
    evecs = [jnp.full((L,), e, dtype=jnp.int32) for e in range(L)]
    cw_i = lax.shift_right_logical(iota, 2)
    cw_j = lax.bitwise_and(iota, 3)
    pr_i = lax.div(iota, jnp.full((L,), 3, jnp.int32))
    pr_j = iota - pr_i * 3
    pr_msk = iota < 9

    for c in range(NCHUNK):
        handles[c].wait()

        def group(g, _, c=c):
            cidv = cid_v[c, pl.ds(g * L, L)]
            for e in range(L):
                b = g * L + e
                bvec = jnp.full((L,), b, dtype=jnp.int32)
                cb16 = lax.shift_left(_take16(cidv, evecs[e]), 4)
                rrow = rrows_v[c, b]
                prow = plsc.load_gather(projtab_v, [cb16 + iota])
                plsc.store_scatter(prjT_v.at[c], [pr_i, pr_j, bvec], prow,
                                   mask=pr_msk)
                acc = None
                for j in range(4):
                    aj = plsc.load_gather(camtab_v, [cb16 + idx_a[j]])
                    term = aj * _take16(rrow, idx_r[j])
                    acc = term if acc is None else acc + term
                plsc.store_scatter(cwT_v.at[c], [cw_i, cw_j, bvec], acc)
            return 0

        lax.fori_loop(0, CHUNK // L, group, 0)

        bb = wid * NCHUNK + c
        pltpu.sync_copy(cwT_v.at[c], cw_out.at[:, bb])
        pltpu.sync_copy(prjT_v.at[c], proj_out.at[:, bb])


def _sc_call(rig, cam, proj, idx4):
    mesh = plsc.VectorSubcoreMesh(core_axis_name="c", subcore_axis_name="s")
    f = pl.kernel(
        _sc_body,
        out_type=[
            jax.ShapeDtypeStruct((4, NW * NCHUNK, 4, CHUNK), jnp.float32),
            jax.ShapeDtypeStruct((3, NW * NCHUNK, 4, CHUNK), jnp.float32),
        ],
        mesh=mesh,
        scratch_types=[
            pltpu.VMEM((NCHUNK, CHUNK), jnp.int32),
            pltpu.VMEM((NCHUNK, CHUNK), jnp.int32),
            pltpu.VMEM((NCHUNK, CHUNK), jnp.int32),
            pltpu.VMEM((NCHUNK, CHUNK, L), jnp.float32),
            pltpu.VMEM((8 * L,), jnp.float32),
            pltpu.VMEM((8 * L,), jnp.float32),
            pltpu.VMEM((NCHUNK, 4, 4, CHUNK), jnp.float32),
            pltpu.VMEM((NCHUNK, 3, 4, CHUNK), jnp.float32),
            pltpu.SemaphoreType.DMA,
        ],
        compiler_params=pltpu.CompilerParams(
            use_tc_tiling_on_sc=False, needs_layout_passes=False),
    )
    return f(rig, cam, proj, idx4)


def kernel(rig_t_world, camera_t_rig, projection, image_idx):
    nf = rig_t_world.shape[0]
    ncam = camera_t_rig.shape[0]
    b = image_idx.shape[0]

    rig = rig_t_world.reshape(nf, 16)
    cam = camera_t_rig.reshape(ncam * 16)
    projpad = jnp.concatenate(
        [projection.reshape(ncam, 9),
         jnp.zeros((ncam, 7), jnp.float32)], axis=1).reshape(ncam * 16)
    idx4 = image_idx.reshape(NW, NCHUNK, CHUNK)

    cwp, prjp = _sc_call(rig, cam, projpad, idx4)

    camera_t_world = cwp.transpose(1, 3, 0, 2).reshape(b, 4, 4)
    proj = prjp[:, :, :3, :].transpose(1, 3, 0, 2).reshape(b, 3, 3)
    return (camera_t_world, proj)
